# baseline (speedup 1.0000x reference)
import jax
import jax.numpy as jnp
from jax.experimental import pallas as pl
from jax.experimental.pallas import tpu as pltpu

_EPS = 1e-06
_HW = 128 * 128
_RIDGE = 0.001 + 2e-06
_NS_ITERS = 8
_INT_MIN = -2147483648
_INT_MAX = 2147483647
_MASK31 = 2147483647


def _float_keys(x):
    i = jax.lax.bitcast_convert_type(x, jnp.int32)
    return jnp.where(i >= 0, i, i ^ _MASK31)


def _keys_to_float(k):
    i = jnp.where(k >= 0, k, k ^ _MASK31)
    return jax.lax.bitcast_convert_type(i, jnp.float32)


def _nested_bisect_i16(k, th, tw, iters):
    c_dim = k.shape[0]
    lo = jnp.full((c_dim, 1, 1), -32768, jnp.int32)
    hi = jnp.full((c_dim, 1, 1), 32767, jnp.int32)

    one = jnp.bfloat16(1)
    zero = jnp.bfloat16(0)

    def body(_, carry):
        lo, hi = carry
        mid = (lo + hi) >> 1
        inner = jnp.sum(jnp.where(k <= mid.astype(jnp.int16), one, zero),
                        axis=1, keepdims=True, dtype=jnp.bfloat16)
        outer = jnp.sum(jnp.where(inner >= jnp.bfloat16(th), one, zero),
                        axis=2, keepdims=True, dtype=jnp.bfloat16)
        ok = outer >= jnp.bfloat16(tw)
        return jnp.where(ok, lo, mid + 1), jnp.where(ok, mid, hi)

    lo, hi = jax.lax.fori_loop(0, iters, body, (lo, hi))
    return lo


def _nested_median_keys(keys):
    c_dim, h_n, w_n = keys.shape
    th = (h_n - 1) // 2 + 1
    tw = (w_n - 1) // 2 + 1

    k_hi = (keys >> 16).astype(jnp.int16)
    hs = _nested_bisect_i16(k_hi, th, tw, 16)

    k_lo = (keys ^ 32768).astype(jnp.int16)
    hs16 = hs.astype(jnp.int16)
    adj = jnp.where(k_hi < hs16, jnp.int16(-32768),
                    jnp.where(k_hi > hs16, jnp.int16(32767), k_lo))
    ls = _nested_bisect_i16(adj, th, tw, 10)

    return (hs << 16) + ((ls & 65535) ^ 32768)


def _stats_kernel(x4_ref, stats_ref, w_ref):
    c_dim = x4_ref.shape[1]
    x4 = x4_ref[0]
    n = float(_HW)

    s1 = jnp.sum(x4, axis=(1, 2), keepdims=True)
    mu = s1 * (1.0 / n)
    d = x4 - mu
    ss = jnp.sum(d * d, axis=(1, 2), keepdims=True)
    var = ss * (1.0 / (n - 1.0))
    inv_sig = jax.lax.rsqrt(var + _EPS)

    keys = _float_keys(x4)
    med = _keys_to_float(_nested_median_keys(keys)[:, 0, :])

    raw = jnp.sum(
        jax.lax.dot_general(x4, x4, (((2,), (2,)), ((1,), (1,))),
                            preferred_element_type=jnp.float32),
        axis=0)
    r = jax.lax.broadcasted_iota(jnp.int32, (c_dim, c_dim), 0)
    c = jax.lax.broadcasted_iota(jnp.int32, (c_dim, c_dim), 1)
    gr = (r >= 16).astype(jnp.int32) + (r >= 32).astype(jnp.int32)
    gc = (c >= 16).astype(jnp.int32) + (c >= 32).astype(jnp.int32)
    same_group = gr == gc
    eye = jnp.where(r == c, 1.0, 0.0)
    a_mat = (jnp.where(same_group, raw * (1.0 / (n + _EPS)), 0.0)
             + _RIDGE * eye)

    nrm = jnp.max(jnp.sum(jnp.abs(a_mat), axis=1))
    y = a_mat * (1.0 / nrm)
    z = eye
    for i in range(_NS_ITERS):
        t = 1.5 * eye - 0.5 * jnp.dot(z, y, preferred_element_type=jnp.float32)
        if i + 1 < _NS_ITERS:
            y = jnp.dot(y, t, preferred_element_type=jnp.float32)
        z = jnp.dot(t, z, preferred_element_type=jnp.float32)
    w_ref[0] = z * jax.lax.rsqrt(nrm)

    stats_ref[0] = jnp.concatenate(
        [mu[:, 0, :], inv_sig[:, 0, :], med,
         jnp.zeros((c_dim, 5), jnp.float32)], axis=1)


def _mix_kernel(perm_ref, lam_ref, xs_ref, xp_ref, ss_ref, sp_ref,
                ws_ref, wp_ref, out_ref):
    del perm_ref
    b = pl.program_id(0)
    lam = lam_ref[b]

    ss = ss_ref[0]
    sp = sp_ref[0]
    mu_s = ss[:, 0:1]
    inv_sig_s = ss[:, 1:2]
    med_s = ss[:, 2:3]
    mu_p = sp[:, 0:1]
    med_p = sp[:, 2:3]

    xcs = xs_ref[0] - mu_s
    xcp = xp_ref[0] - mu_p

    w_mix = jnp.concatenate([ws_ref[0] * lam, wp_ref[0] * (1.0 - lam)],
                            axis=1)
    x2 = jnp.concatenate([xcs, xcp], axis=0)
    gram_mix = jax.lax.dot_general(w_mix, x2, (((1,), (0,)), ((), ())),
                                   preferred_element_type=jnp.float32)

    med_mix = med_s * lam + med_p * (1.0 - lam)
    out_ref[0] = (xcs * inv_sig_s) * gram_mix + med_mix


def kernel(x, lmda, perm):
    b_dim, c_dim, h_dim, w_dim = x.shape
    hw = h_dim * w_dim
    x3 = x.reshape(b_dim, c_dim, hw)
    lam = lmda.reshape(b_dim)

    stats, wmat = pl.pallas_call(
        _stats_kernel,
        grid=(b_dim,),
        in_specs=[
            pl.BlockSpec((1, c_dim, h_dim, w_dim), lambda b: (b, 0, 0, 0)),
        ],
        out_specs=[
            pl.BlockSpec((1, c_dim, 8), lambda b: (b, 0, 0)),
            pl.BlockSpec((1, c_dim, c_dim), lambda b: (b, 0, 0)),
        ],
        out_shape=[
            jax.ShapeDtypeStruct((b_dim, c_dim, 8), jnp.float32),
            jax.ShapeDtypeStruct((b_dim, c_dim, c_dim), jnp.float32),
        ],
        compiler_params=pltpu.CompilerParams(
            dimension_semantics=("parallel",),
            vmem_limit_bytes=48 * 1024 * 1024,
        ),
        name="hgm_stats",
    )(x)

    n_split = 4
    blk = hw // n_split
    out3 = pl.pallas_call(
        _mix_kernel,
        grid_spec=pltpu.PrefetchScalarGridSpec(
            num_scalar_prefetch=2,
            grid=(b_dim, n_split),
            in_specs=[
                pl.BlockSpec((1, c_dim, blk), lambda b, j, pr, lr: (b, 0, j)),
                pl.BlockSpec((1, c_dim, blk),
                             lambda b, j, pr, lr: (pr[b], 0, j)),
                pl.BlockSpec((1, c_dim, 8), lambda b, j, pr, lr: (b, 0, 0)),
                pl.BlockSpec((1, c_dim, 8),
                             lambda b, j, pr, lr: (pr[b], 0, 0)),
                pl.BlockSpec((1, c_dim, c_dim),
                             lambda b, j, pr, lr: (b, 0, 0)),
                pl.BlockSpec((1, c_dim, c_dim),
                             lambda b, j, pr, lr: (pr[b], 0, 0)),
            ],
            out_specs=pl.BlockSpec((1, c_dim, blk),
                                   lambda b, j, pr, lr: (b, 0, j)),
        ),
        out_shape=jax.ShapeDtypeStruct((b_dim, c_dim, hw), jnp.float32),
        compiler_params=pltpu.CompilerParams(
            dimension_semantics=("parallel", "arbitrary"),
            vmem_limit_bytes=48 * 1024 * 1024,
        ),
        name="hgm_mix",
    )(perm, lam, x3, x3, stats, stats, wmat, wmat)

    return out3.reshape(b_dim, c_dim, h_dim, w_dim)


# speedup vs baseline: 1.1797x; 1.1797x over previous
import jax
import jax.numpy as jnp
from jax.experimental import pallas as pl
from jax.experimental.pallas import tpu as pltpu

_EPS = 1e-06
_HW = 128 * 128
_RIDGE = 0.001 + 2e-06
_NS_ITERS = 8
_INT_MIN = -2147483648
_INT_MAX = 2147483647
_MASK31 = 2147483647


def _float_keys(x):
    i = jax.lax.bitcast_convert_type(x, jnp.int32)
    return jnp.where(i >= 0, i, i ^ _MASK31)


def _keys_to_float(k):
    i = jnp.where(k >= 0, k, k ^ _MASK31)
    return jax.lax.bitcast_convert_type(i, jnp.float32)


def _nested_bisect_i16(k, th, tw, iters):
    c_dim = k.shape[0]
    lo = jnp.full((c_dim, 1, 1), -32768, jnp.int32)
    hi = jnp.full((c_dim, 1, 1), 32767, jnp.int32)

    one = jnp.bfloat16(1)
    zero = jnp.bfloat16(0)

    def body(_, carry):
        lo, hi = carry
        mid = (lo + hi) >> 1
        inner = jnp.sum(jnp.where(k <= mid.astype(jnp.int16), one, zero),
                        axis=1, keepdims=True, dtype=jnp.bfloat16)
        outer = jnp.sum(jnp.where(inner >= jnp.bfloat16(th), one, zero),
                        axis=2, keepdims=True, dtype=jnp.bfloat16)
        ok = outer >= jnp.bfloat16(tw)
        return jnp.where(ok, lo, mid + 1), jnp.where(ok, mid, hi)

    lo, hi = jax.lax.fori_loop(0, iters, body, (lo, hi))
    return lo


def _nested_median_keys(keys):
    c_dim, h_n, w_n = keys.shape
    th = (h_n - 1) // 2 + 1
    tw = (w_n - 1) // 2 + 1

    k_hi = (keys >> 16).astype(jnp.int16)
    hs = _nested_bisect_i16(k_hi, th, tw, 16)

    k_lo = (keys ^ 32768).astype(jnp.int16)
    hs16 = hs.astype(jnp.int16)
    adj = jnp.where(k_hi < hs16, jnp.int16(-32768),
                    jnp.where(k_hi > hs16, jnp.int16(32767), k_lo))
    ls = _nested_bisect_i16(adj, th, tw, 10)

    return (hs << 16) + ((ls & 65535) ^ 32768)


def _stats_kernel(x4_ref, x3_ref, stats_ref, w_ref):
    pair, c_dim = x3_ref.shape[0], x3_ref.shape[1]
    c2 = pair * c_dim
    n = float(_HW)

    x4 = x4_ref[...].reshape(c2, x4_ref.shape[2], x4_ref.shape[3])
    x3 = x3_ref[...].reshape(c2, x3_ref.shape[2])

    s1 = jnp.sum(x3, axis=1, keepdims=True)
    mu = s1 * (1.0 / n)
    s2 = jnp.sum(x3 * x3, axis=1, keepdims=True)
    var = (s2 - n * mu * mu) * (1.0 / (n - 1.0))
    inv_sig = jax.lax.rsqrt(var + _EPS)

    keys = _float_keys(x4)
    med = _keys_to_float(_nested_median_keys(keys)[:, 0, :])

    raw = jax.lax.dot_general(x3, x3, (((1,), (1,)), ((), ())),
                              preferred_element_type=jnp.float32)
    r = jax.lax.broadcasted_iota(jnp.int32, (c2, c2), 0)
    c = jax.lax.broadcasted_iota(jnp.int32, (c2, c2), 1)
    rc = r & (c_dim - 1)
    cc = c & (c_dim - 1)
    gr = (r >> 6) * 4 + (rc >= 16).astype(jnp.int32) + (rc >= 32).astype(jnp.int32)
    gc = (c >> 6) * 4 + (cc >= 16).astype(jnp.int32) + (cc >= 32).astype(jnp.int32)
    same_group = gr == gc
    eye = jnp.where(r == c, 1.0, 0.0)
    a_mat = (jnp.where(same_group, raw * (1.0 / (n + _EPS)), 0.0)
             + _RIDGE * eye)

    nrm = jnp.max(jnp.sum(jnp.abs(a_mat), axis=1))
    y = a_mat * (1.0 / nrm)
    z = eye
    for i in range(_NS_ITERS):
        t = 1.5 * eye - 0.5 * jnp.dot(z, y, preferred_element_type=jnp.float32)
        if i + 1 < _NS_ITERS:
            y = jnp.dot(y, t, preferred_element_type=jnp.float32)
        z = jnp.dot(t, z, preferred_element_type=jnp.float32)
    w_full = z * jax.lax.rsqrt(nrm)
    w_ref[0] = w_full[:c_dim, :c_dim]
    w_ref[1] = w_full[c_dim:, c_dim:]

    stats_ref[...] = jnp.concatenate(
        [mu, inv_sig, med, jnp.zeros((c2, 5), jnp.float32)],
        axis=1).reshape(pair, c_dim, 8)


def _mix_kernel(perm_ref, lam_ref, xs_ref, xp_ref, ss_ref, sp_ref,
                ws_ref, wp_ref, out_ref):
    del perm_ref
    b = pl.program_id(0)
    lam = lam_ref[b]

    ss = ss_ref[0]
    sp = sp_ref[0]
    mu_s = ss[:, 0:1]
    inv_sig_s = ss[:, 1:2]
    med_s = ss[:, 2:3]
    mu_p = sp[:, 0:1]
    med_p = sp[:, 2:3]

    xcs = xs_ref[0] - mu_s
    xcp = xp_ref[0] - mu_p

    w_mix = jnp.concatenate([ws_ref[0] * lam, wp_ref[0] * (1.0 - lam)],
                            axis=1)
    x2 = jnp.concatenate([xcs, xcp], axis=0)
    gram_mix = jax.lax.dot_general(w_mix, x2, (((1,), (0,)), ((), ())),
                                   preferred_element_type=jnp.float32)

    med_mix = med_s * lam + med_p * (1.0 - lam)
    out_ref[0] = (xcs * inv_sig_s) * gram_mix + med_mix


def kernel(x, lmda, perm):
    b_dim, c_dim, h_dim, w_dim = x.shape
    hw = h_dim * w_dim
    x3 = x.reshape(b_dim, c_dim, hw)
    lam = lmda.reshape(b_dim)

    stats, wmat = pl.pallas_call(
        _stats_kernel,
        grid=(b_dim // 2,),
        in_specs=[
            pl.BlockSpec((2, c_dim, h_dim, w_dim), lambda b: (b, 0, 0, 0)),
            pl.BlockSpec((2, c_dim, hw), lambda b: (b, 0, 0)),
        ],
        out_specs=[
            pl.BlockSpec((2, c_dim, 8), lambda b: (b, 0, 0)),
            pl.BlockSpec((2, c_dim, c_dim), lambda b: (b, 0, 0)),
        ],
        out_shape=[
            jax.ShapeDtypeStruct((b_dim, c_dim, 8), jnp.float32),
            jax.ShapeDtypeStruct((b_dim, c_dim, c_dim), jnp.float32),
        ],
        compiler_params=pltpu.CompilerParams(
            dimension_semantics=("parallel",),
            vmem_limit_bytes=56 * 1024 * 1024,
        ),
        name="hgm_stats",
    )(x, x3)

    n_split = 4
    blk = hw // n_split
    out3 = pl.pallas_call(
        _mix_kernel,
        grid_spec=pltpu.PrefetchScalarGridSpec(
            num_scalar_prefetch=2,
            grid=(b_dim, n_split),
            in_specs=[
                pl.BlockSpec((1, c_dim, blk), lambda b, j, pr, lr: (b, 0, j)),
                pl.BlockSpec((1, c_dim, blk),
                             lambda b, j, pr, lr: (pr[b], 0, j)),
                pl.BlockSpec((1, c_dim, 8), lambda b, j, pr, lr: (b, 0, 0)),
                pl.BlockSpec((1, c_dim, 8),
                             lambda b, j, pr, lr: (pr[b], 0, 0)),
                pl.BlockSpec((1, c_dim, c_dim),
                             lambda b, j, pr, lr: (b, 0, 0)),
                pl.BlockSpec((1, c_dim, c_dim),
                             lambda b, j, pr, lr: (pr[b], 0, 0)),
            ],
            out_specs=pl.BlockSpec((1, c_dim, blk),
                                   lambda b, j, pr, lr: (b, 0, j)),
        ),
        out_shape=jax.ShapeDtypeStruct((b_dim, c_dim, hw), jnp.float32),
        compiler_params=pltpu.CompilerParams(
            dimension_semantics=("parallel", "arbitrary"),
            vmem_limit_bytes=48 * 1024 * 1024,
        ),
        name="hgm_mix",
    )(perm, lam, x3, x3, stats, stats, wmat, wmat)

    return out3.reshape(b_dim, c_dim, h_dim, w_dim)


# speedup vs baseline: 1.2656x; 1.0728x over previous
import jax
import jax.numpy as jnp
from jax.experimental import pallas as pl
from jax.experimental.pallas import tpu as pltpu

_EPS = 1e-06
_HW = 128 * 128
_RIDGE = 0.001 + 2e-06
_NS_ITERS = 8
_INT_MIN = -2147483648
_INT_MAX = 2147483647
_MASK31 = 2147483647


def _float_keys(x):
    i = jax.lax.bitcast_convert_type(x, jnp.int32)
    return jnp.where(i >= 0, i, i ^ _MASK31)


def _keys_to_float(k):
    i = jnp.where(k >= 0, k, k ^ _MASK31)
    return jax.lax.bitcast_convert_type(i, jnp.float32)


def _nested_bisect_i16(k, th, tw, iters):
    c_dim = k.shape[0]
    lo = jnp.full((c_dim, 1, 1), -32768, jnp.int32)
    hi = jnp.full((c_dim, 1, 1), 32767, jnp.int32)

    one = jnp.bfloat16(1)
    zero = jnp.bfloat16(0)

    def body(_, carry):
        lo, hi = carry
        mid = (lo + hi) >> 1
        inner = jnp.sum(jnp.where(k <= mid.astype(jnp.int16), one, zero),
                        axis=1, keepdims=True, dtype=jnp.bfloat16)
        outer = jnp.sum(jnp.where(inner >= jnp.bfloat16(th), one, zero),
                        axis=2, keepdims=True, dtype=jnp.bfloat16)
        ok = outer >= jnp.bfloat16(tw)
        return jnp.where(ok, lo, mid + 1), jnp.where(ok, mid, hi)

    lo, hi = jax.lax.fori_loop(0, iters, body, (lo, hi))
    return lo


def _nested_median_keys(keys):
    c_dim, h_n, w_n = keys.shape
    th = (h_n - 1) // 2 + 1
    tw = (w_n - 1) // 2 + 1

    k_hi = (keys >> 16).astype(jnp.int16)
    hs = _nested_bisect_i16(k_hi, th, tw, 16)

    k_lo = (keys ^ 32768).astype(jnp.int16)
    hs16 = hs.astype(jnp.int16)
    adj = jnp.where(k_hi < hs16, jnp.int16(-32768),
                    jnp.where(k_hi > hs16, jnp.int16(32767), k_lo))
    ls = _nested_bisect_i16(adj, th, tw, 6)

    return (hs << 16) + ((ls & 65535) ^ 32768)


def _stats_kernel(x4_ref, x3_ref, stats_ref, w_ref):
    pair, c_dim = x3_ref.shape[0], x3_ref.shape[1]
    c2 = pair * c_dim
    n = float(_HW)

    x4 = x4_ref[...].reshape(c2, x4_ref.shape[2], x4_ref.shape[3])
    x3 = x3_ref[...].reshape(c2, x3_ref.shape[2])

    s1 = jnp.sum(x3, axis=1, keepdims=True)
    mu = s1 * (1.0 / n)
    s2 = jnp.sum(x3 * x3, axis=1, keepdims=True)
    var = (s2 - n * mu * mu) * (1.0 / (n - 1.0))
    inv_sig = jax.lax.rsqrt(var + _EPS)

    keys = _float_keys(x4)
    med = _keys_to_float(_nested_median_keys(keys)[:, 0, :])

    raw = jax.lax.dot_general(x3, x3, (((1,), (1,)), ((), ())),
                              preferred_element_type=jnp.float32)
    r = jax.lax.broadcasted_iota(jnp.int32, (c2, c2), 0)
    c = jax.lax.broadcasted_iota(jnp.int32, (c2, c2), 1)
    rc = r & (c_dim - 1)
    cc = c & (c_dim - 1)
    gr = (r >> 6) * 4 + (rc >= 16).astype(jnp.int32) + (rc >= 32).astype(jnp.int32)
    gc = (c >> 6) * 4 + (cc >= 16).astype(jnp.int32) + (cc >= 32).astype(jnp.int32)
    same_group = gr == gc
    eye = jnp.where(r == c, 1.0, 0.0)
    a_mat = (jnp.where(same_group, raw * (1.0 / (n + _EPS)), 0.0)
             + _RIDGE * eye)

    nrm = jnp.max(jnp.sum(jnp.abs(a_mat), axis=1))
    y = a_mat * (1.0 / nrm)
    z = eye
    for i in range(_NS_ITERS):
        t = 1.5 * eye - 0.5 * jnp.dot(z, y, preferred_element_type=jnp.float32)
        if i + 1 < _NS_ITERS:
            y = jnp.dot(y, t, preferred_element_type=jnp.float32)
        z = jnp.dot(t, z, preferred_element_type=jnp.float32)
    w_full = z * jax.lax.rsqrt(nrm)
    w_ref[0] = w_full[:c_dim, :c_dim]
    w_ref[1] = w_full[c_dim:, c_dim:]

    stats_ref[...] = jnp.concatenate(
        [mu, inv_sig, med, jnp.zeros((c2, 5), jnp.float32)],
        axis=1).reshape(pair, c_dim, 8)


def _mix_kernel(perm_ref, lam_ref, xs_ref, xp_ref, ss_ref, sp_ref,
                ws_ref, wp_ref, out_ref):
    del perm_ref
    b = pl.program_id(0)
    lam = lam_ref[b]

    ss = ss_ref[0]
    sp = sp_ref[0]
    mu_s = ss[:, 0:1]
    inv_sig_s = ss[:, 1:2]
    med_s = ss[:, 2:3]
    mu_p = sp[:, 0:1]
    med_p = sp[:, 2:3]

    xcs = xs_ref[0] - mu_s
    xcp = xp_ref[0] - mu_p

    w_mix = jnp.concatenate([ws_ref[0] * lam, wp_ref[0] * (1.0 - lam)],
                            axis=1)
    x2 = jnp.concatenate([xcs, xcp], axis=0)
    gram_mix = jax.lax.dot_general(w_mix, x2, (((1,), (0,)), ((), ())),
                                   preferred_element_type=jnp.float32)

    med_mix = med_s * lam + med_p * (1.0 - lam)
    out_ref[0] = (xcs * inv_sig_s) * gram_mix + med_mix


def kernel(x, lmda, perm):
    b_dim, c_dim, h_dim, w_dim = x.shape
    hw = h_dim * w_dim
    x3 = x.reshape(b_dim, c_dim, hw)
    lam = lmda.reshape(b_dim)

    stats, wmat = pl.pallas_call(
        _stats_kernel,
        grid=(b_dim // 2,),
        in_specs=[
            pl.BlockSpec((2, c_dim, h_dim, w_dim), lambda b: (b, 0, 0, 0)),
            pl.BlockSpec((2, c_dim, hw), lambda b: (b, 0, 0)),
        ],
        out_specs=[
            pl.BlockSpec((2, c_dim, 8), lambda b: (b, 0, 0)),
            pl.BlockSpec((2, c_dim, c_dim), lambda b: (b, 0, 0)),
        ],
        out_shape=[
            jax.ShapeDtypeStruct((b_dim, c_dim, 8), jnp.float32),
            jax.ShapeDtypeStruct((b_dim, c_dim, c_dim), jnp.float32),
        ],
        compiler_params=pltpu.CompilerParams(
            dimension_semantics=("parallel",),
            vmem_limit_bytes=56 * 1024 * 1024,
        ),
        name="hgm_stats",
    )(x, x3)

    n_split = 4
    blk = hw // n_split
    out3 = pl.pallas_call(
        _mix_kernel,
        grid_spec=pltpu.PrefetchScalarGridSpec(
            num_scalar_prefetch=2,
            grid=(b_dim, n_split),
            in_specs=[
                pl.BlockSpec((1, c_dim, blk), lambda b, j, pr, lr: (b, 0, j)),
                pl.BlockSpec((1, c_dim, blk),
                             lambda b, j, pr, lr: (pr[b], 0, j)),
                pl.BlockSpec((1, c_dim, 8), lambda b, j, pr, lr: (b, 0, 0)),
                pl.BlockSpec((1, c_dim, 8),
                             lambda b, j, pr, lr: (pr[b], 0, 0)),
                pl.BlockSpec((1, c_dim, c_dim),
                             lambda b, j, pr, lr: (b, 0, 0)),
                pl.BlockSpec((1, c_dim, c_dim),
                             lambda b, j, pr, lr: (pr[b], 0, 0)),
            ],
            out_specs=pl.BlockSpec((1, c_dim, blk),
                                   lambda b, j, pr, lr: (b, 0, j)),
        ),
        out_shape=jax.ShapeDtypeStruct((b_dim, c_dim, hw), jnp.float32),
        compiler_params=pltpu.CompilerParams(
            dimension_semantics=("parallel", "arbitrary"),
            vmem_limit_bytes=48 * 1024 * 1024,
        ),
        name="hgm_mix",
    )(perm, lam, x3, x3, stats, stats, wmat, wmat)

    return out3.reshape(b_dim, c_dim, h_dim, w_dim)


# speedup vs baseline: 1.3566x; 1.0719x over previous
import jax
import jax.numpy as jnp
from jax.experimental import pallas as pl
from jax.experimental.pallas import tpu as pltpu

_EPS = 1e-06
_HW = 128 * 128
_RIDGE = 0.001 + 2e-06
_NS_ITERS = 8
_INT_MIN = -2147483648
_INT_MAX = 2147483647
_MASK31 = 2147483647


def _float_keys(x):
    i = jax.lax.bitcast_convert_type(x, jnp.int32)
    return jnp.where(i >= 0, i, i ^ _MASK31)


def _keys_to_float(k):
    i = jnp.where(k >= 0, k, k ^ _MASK31)
    return jax.lax.bitcast_convert_type(i, jnp.float32)


def _nested_bisect_i16(k, th, tw, iters):
    c_dim = k.shape[0]
    lo = jnp.full((c_dim, 1, 1), -32768, jnp.int32)
    hi = jnp.full((c_dim, 1, 1), 32767, jnp.int32)

    one = jnp.bfloat16(1)
    zero = jnp.bfloat16(0)

    def body(_, carry):
        lo, hi = carry
        mid = (lo + hi) >> 1
        inner = jnp.sum(jnp.where(k <= mid.astype(jnp.int16), one, zero),
                        axis=1, keepdims=True, dtype=jnp.bfloat16)
        outer = jnp.sum(jnp.where(inner >= jnp.bfloat16(th), one, zero),
                        axis=2, keepdims=True, dtype=jnp.bfloat16)
        ok = outer >= jnp.bfloat16(tw)
        return jnp.where(ok, lo, mid + 1), jnp.where(ok, mid, hi)

    lo, hi = jax.lax.fori_loop(0, iters, body, (lo, hi))
    return lo


def _nested_median_keys(keys):
    c_dim, h_n, w_n = keys.shape
    th = (h_n - 1) // 2 + 1
    tw = (w_n - 1) // 2 + 1

    k_hi = (keys >> 16).astype(jnp.int16)
    hs = _nested_bisect_i16(k_hi, th, tw, 16)

    k_lo = (keys ^ 32768).astype(jnp.int16)
    hs16 = hs.astype(jnp.int16)
    adj = jnp.where(k_hi < hs16, jnp.int16(-32768),
                    jnp.where(k_hi > hs16, jnp.int16(32767), k_lo))
    ls = _nested_bisect_i16(adj, th, tw, 6)

    return (hs << 16) + ((ls & 65535) ^ 32768)


def _stats_kernel(x4_ref, x3_ref, stats_ref, w_ref):
    pair, c_dim = x3_ref.shape[0], x3_ref.shape[1]
    c2 = pair * c_dim
    n = float(_HW)

    x4 = x4_ref[...].reshape(c2, x4_ref.shape[2], x4_ref.shape[3])
    x3 = x3_ref[...].reshape(c2, x3_ref.shape[2])

    s1 = jnp.sum(x3, axis=1, keepdims=True)
    mu = s1 * (1.0 / n)
    s2 = jnp.sum(x3 * x3, axis=1, keepdims=True)
    var = (s2 - n * mu * mu) * (1.0 / (n - 1.0))
    inv_sig = jax.lax.rsqrt(var + _EPS)

    keys = _float_keys(x4)
    med = _keys_to_float(_nested_median_keys(keys)[:, 0, :])

    raw = jax.lax.dot_general(x3, x3, (((1,), (1,)), ((), ())),
                              preferred_element_type=jnp.float32)
    r = jax.lax.broadcasted_iota(jnp.int32, (c2, c2), 0)
    c = jax.lax.broadcasted_iota(jnp.int32, (c2, c2), 1)
    rc = r & (c_dim - 1)
    cc = c & (c_dim - 1)
    gr = (r >> 6) * 4 + (rc >= 16).astype(jnp.int32) + (rc >= 32).astype(jnp.int32)
    gc = (c >> 6) * 4 + (cc >= 16).astype(jnp.int32) + (cc >= 32).astype(jnp.int32)
    same_group = gr == gc
    eye = jnp.where(r == c, 1.0, 0.0)
    a_mat = (jnp.where(same_group, raw * (1.0 / (n + _EPS)), 0.0)
             + _RIDGE * eye)

    nrm = jnp.max(jnp.sum(jnp.abs(a_mat), axis=1))
    y = a_mat * (1.0 / nrm)
    z = eye
    for i in range(_NS_ITERS):
        t = 1.5 * eye - 0.5 * jnp.dot(z, y, preferred_element_type=jnp.float32)
        if i + 1 < _NS_ITERS:
            y = jnp.dot(y, t, preferred_element_type=jnp.float32)
        z = jnp.dot(t, z, preferred_element_type=jnp.float32)
    w_full = z * jax.lax.rsqrt(nrm)
    w_ref[0] = w_full[:c_dim, :c_dim]
    w_ref[1] = w_full[c_dim:, c_dim:]

    stats_ref[...] = jnp.concatenate(
        [mu, inv_sig, med, jnp.zeros((c2, 5), jnp.float32)],
        axis=1).reshape(pair, c_dim, 8)


def _mix_kernel(perm_ref, lam_ref, xs_ref, xp_ref, ss_ref, sp_ref,
                ws_ref, wp_ref, out_ref):
    del perm_ref
    b = pl.program_id(0)
    lam = lam_ref[b]

    ss = ss_ref[0]
    sp = sp_ref[0]
    mu_s = ss[:, 0:1, None]
    inv_sig_s = ss[:, 1:2, None]
    med_s = ss[:, 2:3, None]
    mu_p = sp[:, 0:1, None]
    med_p = sp[:, 2:3, None]

    xcs = xs_ref[0] - mu_s
    xcp = xp_ref[0] - mu_p

    w_mix = jnp.concatenate([ws_ref[0] * lam, wp_ref[0] * (1.0 - lam)],
                            axis=1)
    x2 = jnp.concatenate([xcs, xcp], axis=0)
    gram_mix = jax.lax.dot_general(w_mix, x2, (((1,), (0,)), ((), ())),
                                   preferred_element_type=jnp.float32)

    med_mix = med_s * lam + med_p * (1.0 - lam)
    out_ref[0] = (xcs * inv_sig_s) * gram_mix + med_mix


def kernel(x, lmda, perm):
    b_dim, c_dim, h_dim, w_dim = x.shape
    hw = h_dim * w_dim
    x3 = x.reshape(b_dim, c_dim, hw)
    lam = lmda.reshape(b_dim)

    stats, wmat = pl.pallas_call(
        _stats_kernel,
        grid=(b_dim // 2,),
        in_specs=[
            pl.BlockSpec((2, c_dim, h_dim, w_dim), lambda b: (b, 0, 0, 0)),
            pl.BlockSpec((2, c_dim, hw), lambda b: (b, 0, 0)),
        ],
        out_specs=[
            pl.BlockSpec((2, c_dim, 8), lambda b: (b, 0, 0)),
            pl.BlockSpec((2, c_dim, c_dim), lambda b: (b, 0, 0)),
        ],
        out_shape=[
            jax.ShapeDtypeStruct((b_dim, c_dim, 8), jnp.float32),
            jax.ShapeDtypeStruct((b_dim, c_dim, c_dim), jnp.float32),
        ],
        compiler_params=pltpu.CompilerParams(
            dimension_semantics=("parallel",),
            vmem_limit_bytes=56 * 1024 * 1024,
        ),
        name="hgm_stats",
    )(x, x3)

    n_split = 4
    hb = h_dim // n_split
    out = pl.pallas_call(
        _mix_kernel,
        grid_spec=pltpu.PrefetchScalarGridSpec(
            num_scalar_prefetch=2,
            grid=(b_dim, n_split),
            in_specs=[
                pl.BlockSpec((1, c_dim, hb, w_dim),
                             lambda b, j, pr, lr: (b, 0, j, 0)),
                pl.BlockSpec((1, c_dim, hb, w_dim),
                             lambda b, j, pr, lr: (pr[b], 0, j, 0)),
                pl.BlockSpec((1, c_dim, 8), lambda b, j, pr, lr: (b, 0, 0)),
                pl.BlockSpec((1, c_dim, 8),
                             lambda b, j, pr, lr: (pr[b], 0, 0)),
                pl.BlockSpec((1, c_dim, c_dim),
                             lambda b, j, pr, lr: (b, 0, 0)),
                pl.BlockSpec((1, c_dim, c_dim),
                             lambda b, j, pr, lr: (pr[b], 0, 0)),
            ],
            out_specs=pl.BlockSpec((1, c_dim, hb, w_dim),
                                   lambda b, j, pr, lr: (b, 0, j, 0)),
        ),
        out_shape=jax.ShapeDtypeStruct((b_dim, c_dim, h_dim, w_dim),
                                       jnp.float32),
        compiler_params=pltpu.CompilerParams(
            dimension_semantics=("parallel", "arbitrary"),
            vmem_limit_bytes=48 * 1024 * 1024,
        ),
        name="hgm_mix",
    )(perm, lam, x, x, stats, stats, wmat, wmat)

    return out


# speedup vs baseline: 1.4471x; 1.0667x over previous
import jax
import jax.numpy as jnp
from jax.experimental import pallas as pl
from jax.experimental.pallas import tpu as pltpu

_EPS = 1e-06
_HW = 128 * 128
_RIDGE = 0.001 + 2e-06
_NS_ITERS = 8
_INT_MIN = -2147483648
_INT_MAX = 2147483647
_MASK31 = 2147483647


def _float_keys(x):
    i = jax.lax.bitcast_convert_type(x, jnp.int32)
    return jnp.where(i >= 0, i, i ^ _MASK31)


def _keys_to_float(k):
    i = jnp.where(k >= 0, k, k ^ _MASK31)
    return jax.lax.bitcast_convert_type(i, jnp.float32)


def _nested_bisect_i16(k, th, tw, iters):
    c_dim = k.shape[0]
    lo = jnp.full((c_dim, 1, 1), -32768, jnp.int32)
    hi = jnp.full((c_dim, 1, 1), 32767, jnp.int32)

    one = jnp.bfloat16(1)
    zero = jnp.bfloat16(0)

    def body(_, carry):
        lo, hi = carry
        mid = (lo + hi) >> 1
        inner = jnp.sum(jnp.where(k <= mid.astype(jnp.int16), one, zero),
                        axis=1, keepdims=True, dtype=jnp.bfloat16)
        outer = jnp.sum(jnp.where(inner >= jnp.bfloat16(th), one, zero),
                        axis=2, keepdims=True, dtype=jnp.bfloat16)
        ok = outer >= jnp.bfloat16(tw)
        return jnp.where(ok, lo, mid + 1), jnp.where(ok, mid, hi)

    lo, hi = jax.lax.fori_loop(0, iters, body, (lo, hi))
    return lo


def _nested_median_keys(keys):
    c_dim, h_n, w_n = keys.shape
    th = (h_n - 1) // 2 + 1
    tw = (w_n - 1) // 2 + 1

    k_hi = (keys >> 16).astype(jnp.int16)
    hs = _nested_bisect_i16(k_hi, th, tw, 16)

    k_lo = (keys ^ 32768).astype(jnp.int16)
    hs16 = hs.astype(jnp.int16)
    adj = jnp.where(k_hi < hs16, jnp.int16(-32768),
                    jnp.where(k_hi > hs16, jnp.int16(32767), k_lo))
    ls = _nested_bisect_i16(adj, th, tw, 6)

    return (hs << 16) + ((ls & 65535) ^ 32768)


def _stats_kernel(x4_ref, stats_ref, w_ref):
    pair, c_dim = x4_ref.shape[0], x4_ref.shape[1]
    c2 = pair * c_dim
    n = float(_HW)

    x4 = x4_ref[...].reshape(c2, x4_ref.shape[2], x4_ref.shape[3])

    s1 = jnp.sum(x4, axis=(1, 2), keepdims=True)[:, 0, :]
    mu = s1 * (1.0 / n)
    s2 = jnp.sum(x4 * x4, axis=(1, 2), keepdims=True)[:, 0, :]
    var = (s2 - n * mu * mu) * (1.0 / (n - 1.0))
    inv_sig = jax.lax.rsqrt(var + _EPS)

    keys = _float_keys(x4)
    med = _keys_to_float(_nested_median_keys(keys)[:, 0, :])

    raw = jnp.sum(
        jax.lax.dot_general(x4, x4, (((2,), (2,)), ((1,), (1,))),
                            preferred_element_type=jnp.float32),
        axis=0)
    r = jax.lax.broadcasted_iota(jnp.int32, (c2, c2), 0)
    c = jax.lax.broadcasted_iota(jnp.int32, (c2, c2), 1)
    rc = r & (c_dim - 1)
    cc = c & (c_dim - 1)
    gr = (r >> 6) * 4 + (rc >= 16).astype(jnp.int32) + (rc >= 32).astype(jnp.int32)
    gc = (c >> 6) * 4 + (cc >= 16).astype(jnp.int32) + (cc >= 32).astype(jnp.int32)
    same_group = gr == gc
    eye = jnp.where(r == c, 1.0, 0.0)
    a_mat = (jnp.where(same_group, raw * (1.0 / (n + _EPS)), 0.0)
             + _RIDGE * eye)

    nrm = jnp.max(jnp.sum(jnp.abs(a_mat), axis=1))
    y = a_mat * (1.0 / nrm)
    z = eye
    for i in range(_NS_ITERS):
        t = 1.5 * eye - 0.5 * jnp.dot(z, y, preferred_element_type=jnp.float32)
        if i + 1 < _NS_ITERS:
            y = jnp.dot(y, t, preferred_element_type=jnp.float32)
        z = jnp.dot(t, z, preferred_element_type=jnp.float32)
    w_full = z * jax.lax.rsqrt(nrm)
    w_ref[0] = w_full[:c_dim, :c_dim]
    w_ref[1] = w_full[c_dim:, c_dim:]

    stats_ref[...] = jnp.concatenate(
        [mu, inv_sig, med, jnp.zeros((c2, 5), jnp.float32)],
        axis=1).reshape(pair, c_dim, 8)


def _mix_kernel(perm_ref, lam_ref, xs_ref, xp_ref, ss_ref, sp_ref,
                ws_ref, wp_ref, out_ref):
    del perm_ref
    b = pl.program_id(0)
    lam = lam_ref[b]

    ss = ss_ref[0]
    sp = sp_ref[0]
    mu_s = ss[:, 0:1, None]
    inv_sig_s = ss[:, 1:2, None]
    med_s = ss[:, 2:3, None]
    mu_p = sp[:, 0:1, None]
    med_p = sp[:, 2:3, None]

    xcs = xs_ref[0] - mu_s
    xcp = xp_ref[0] - mu_p

    w_mix = jnp.concatenate([ws_ref[0] * lam, wp_ref[0] * (1.0 - lam)],
                            axis=1)
    x2 = jnp.concatenate([xcs, xcp], axis=0)
    gram_mix = jax.lax.dot_general(w_mix, x2, (((1,), (0,)), ((), ())),
                                   preferred_element_type=jnp.float32)

    med_mix = med_s * lam + med_p * (1.0 - lam)
    out_ref[0] = (xcs * inv_sig_s) * gram_mix + med_mix


def kernel(x, lmda, perm):
    b_dim, c_dim, h_dim, w_dim = x.shape
    lam = lmda.reshape(b_dim)

    stats, wmat = pl.pallas_call(
        _stats_kernel,
        grid=(b_dim // 2,),
        in_specs=[
            pl.BlockSpec((2, c_dim, h_dim, w_dim), lambda b: (b, 0, 0, 0)),
        ],
        out_specs=[
            pl.BlockSpec((2, c_dim, 8), lambda b: (b, 0, 0)),
            pl.BlockSpec((2, c_dim, c_dim), lambda b: (b, 0, 0)),
        ],
        out_shape=[
            jax.ShapeDtypeStruct((b_dim, c_dim, 8), jnp.float32),
            jax.ShapeDtypeStruct((b_dim, c_dim, c_dim), jnp.float32),
        ],
        compiler_params=pltpu.CompilerParams(
            dimension_semantics=("parallel",),
            vmem_limit_bytes=56 * 1024 * 1024,
        ),
        name="hgm_stats",
    )(x)

    n_split = 4
    hb = h_dim // n_split
    out = pl.pallas_call(
        _mix_kernel,
        grid_spec=pltpu.PrefetchScalarGridSpec(
            num_scalar_prefetch=2,
            grid=(b_dim, n_split),
            in_specs=[
                pl.BlockSpec((1, c_dim, hb, w_dim),
                             lambda b, j, pr, lr: (b, 0, j, 0)),
                pl.BlockSpec((1, c_dim, hb, w_dim),
                             lambda b, j, pr, lr: (pr[b], 0, j, 0)),
                pl.BlockSpec((1, c_dim, 8), lambda b, j, pr, lr: (b, 0, 0)),
                pl.BlockSpec((1, c_dim, 8),
                             lambda b, j, pr, lr: (pr[b], 0, 0)),
                pl.BlockSpec((1, c_dim, c_dim),
                             lambda b, j, pr, lr: (b, 0, 0)),
                pl.BlockSpec((1, c_dim, c_dim),
                             lambda b, j, pr, lr: (pr[b], 0, 0)),
            ],
            out_specs=pl.BlockSpec((1, c_dim, hb, w_dim),
                                   lambda b, j, pr, lr: (b, 0, j, 0)),
        ),
        out_shape=jax.ShapeDtypeStruct((b_dim, c_dim, h_dim, w_dim),
                                       jnp.float32),
        compiler_params=pltpu.CompilerParams(
            dimension_semantics=("parallel", "arbitrary"),
            vmem_limit_bytes=48 * 1024 * 1024,
        ),
        name="hgm_mix",
    )(perm, lam, x, x, stats, stats, wmat, wmat)

    return out
